# baseline (speedup 1.0000x reference)
import functools

import jax
import jax.numpy as jnp
from jax import lax
from jax.experimental import pallas as pl
from jax.experimental.pallas import tpu as pltpu


def _argmin_body(k_total, zsq_ref, esq_ref, z_ref, e_ref, out_ref, best_val):
    n = pl.program_id(0)
    m = pl.program_id(1)
    bn = e_ref.shape[0]
    bm = z_ref.shape[0]
    mm = lax.dot_general(e_ref[...].astype(jnp.bfloat16),
                         z_ref[...].astype(jnp.bfloat16),
                         (((1,), (1,)), ((), ())),
                         preferred_element_type=jnp.float32)
    dist = (esq_ref[...] + zsq_ref[...]) - 2.0 * mm
    rows = lax.broadcasted_iota(jnp.int32, (bn, bm), 0) + n * bn
    dist = jnp.where(rows < k_total, dist, jnp.float32(jnp.inf))
    vmin = jnp.min(dist, axis=0, keepdims=True)
    imin = jnp.min(jnp.where(dist == vmin, rows, jnp.int32(2**30)),
                   axis=0, keepdims=True)
    vmin_r = vmin.astype(jnp.bfloat16).astype(jnp.float32)

    @pl.when(n == 0)
    def _init():
        best_val[pl.ds(m, 1), :] = vmin_r
        out_ref[pl.ds(m, 1), :] = imin

    @pl.when(n > 0)
    def _update():
        bv = best_val[pl.ds(m, 1), :]
        bi = out_ref[pl.ds(m, 1), :]
        better = vmin < bv
        newv = jnp.where(better, vmin, bv)
        best_val[pl.ds(m, 1), :] = newv.astype(jnp.bfloat16).astype(jnp.float32)
        out_ref[pl.ds(m, 1), :] = jnp.where(better, imin, bi)


def _fused_argmin(flat_z, embedding, zsq_row, esq_col, bm, bn):
    m_total, d = flat_z.shape
    k_total = embedding.shape[0]
    mt = m_total // bm
    nt = -(-k_total // bn)
    out2d = pl.pallas_call(
        functools.partial(_argmin_body, k_total),
        grid=(nt, mt),
        in_specs=[
            pl.BlockSpec((1, bm), lambda n, m: (0, m)),
            pl.BlockSpec((bn, 1), lambda n, m: (n, 0)),
            pl.BlockSpec((bm, d), lambda n, m: (m, 0)),
            pl.BlockSpec((bn, d), lambda n, m: (n, 0)),
        ],
        out_specs=pl.BlockSpec((mt, bm), lambda n, m: (0, 0)),
        out_shape=jax.ShapeDtypeStruct((mt, bm), jnp.int32),
        scratch_shapes=[
            pltpu.VMEM((mt, bm), jnp.float32),
        ],
        compiler_params=pltpu.CompilerParams(
            dimension_semantics=("arbitrary", "arbitrary")),
    )(zsq_row, esq_col, flat_z, embedding)
    return out2d.reshape(m_total)


def kernel(z, embedding):
    b, d, h, w = z.shape
    k_total = embedding.shape[0]
    flat_z = jnp.transpose(z, (0, 2, 3, 1)).reshape(-1, d)
    zsq = jnp.sum(flat_z ** 2, axis=1, keepdims=True)
    esq = jnp.sum(embedding ** 2, axis=1)

    encoding_indices = _fused_argmin(flat_z, embedding,
                                     zsq.T, esq[:, None],
                                     bm=512, bn=2736)

    quantized_flat = jnp.take(embedding, encoding_indices, axis=0)
    counts = jnp.bincount(encoding_indices, length=k_total).astype(flat_z.dtype)

    z_q_flat = flat_z + (quantized_flat - flat_z)
    z_q = jnp.transpose(z_q_flat.reshape(b, h, w, d), (0, 3, 1, 2))
    loss = jnp.mean((z_q_flat - flat_z) ** 2)
    indices = encoding_indices.reshape(b, h, w)
    avg_probs = counts / (b * h * w)
    perplexity = jnp.exp(-jnp.sum(avg_probs * jnp.log(avg_probs + 1e-10)))
    used_codes = (counts > 0).astype(jnp.float32)
    return (z_q, loss, indices, perplexity, used_codes)


# speedup vs baseline: 1.0231x; 1.0231x over previous
import functools

import jax
import jax.numpy as jnp
from jax import lax
from jax.experimental import pallas as pl
from jax.experimental.pallas import tpu as pltpu


def _argmin_body(zsq_ref, esq_ref, z_ref, e_ref, out_ref, best_val):
    n = pl.program_id(0)
    m = pl.program_id(1)
    bn = e_ref.shape[0]
    bm = z_ref.shape[0]
    mm = lax.dot_general(e_ref[...].astype(jnp.bfloat16),
                         z_ref[...].astype(jnp.bfloat16),
                         (((1,), (1,)), ((), ())),
                         preferred_element_type=jnp.float32)
    dist = (esq_ref[...] + zsq_ref[...]) - mm
    rows = lax.broadcasted_iota(jnp.int32, (bn, bm), 0) + n * bn
    vmin = jnp.min(dist, axis=0, keepdims=True)
    imin = jnp.min(jnp.where(dist == vmin, rows, jnp.int32(2**30)),
                   axis=0, keepdims=True)
    vmin_r = vmin.astype(jnp.bfloat16).astype(jnp.float32)

    @pl.when(n == 0)
    def _init():
        best_val[pl.ds(m, 1), :] = vmin_r
        out_ref[pl.ds(m, 1), :] = imin

    @pl.when(n > 0)
    def _update():
        bv = best_val[pl.ds(m, 1), :]
        bi = out_ref[pl.ds(m, 1), :]
        better = vmin < bv
        newv = jnp.where(better, vmin, bv)
        best_val[pl.ds(m, 1), :] = newv.astype(jnp.bfloat16).astype(jnp.float32)
        out_ref[pl.ds(m, 1), :] = jnp.where(better, imin, bi)


def _fused_argmin(flat_z, embedding, zsq_row, esq_col, bm, bn):
    m_total, d = flat_z.shape
    k_pad = embedding.shape[0]
    mt = m_total // bm
    nt = k_pad // bn
    out2d = pl.pallas_call(
        _argmin_body,
        grid=(nt, mt),
        in_specs=[
            pl.BlockSpec((1, bm), lambda n, m: (0, m)),
            pl.BlockSpec((bn, 1), lambda n, m: (n, 0)),
            pl.BlockSpec((bm, d), lambda n, m: (m, 0)),
            pl.BlockSpec((bn, d), lambda n, m: (n, 0)),
        ],
        out_specs=pl.BlockSpec((mt, bm), lambda n, m: (0, 0)),
        out_shape=jax.ShapeDtypeStruct((mt, bm), jnp.int32),
        scratch_shapes=[
            pltpu.VMEM((mt, bm), jnp.float32),
        ],
        compiler_params=pltpu.CompilerParams(
            dimension_semantics=("arbitrary", "arbitrary")),
    )(zsq_row, esq_col, flat_z, embedding)
    return out2d.reshape(m_total)


def kernel(z, embedding):
    b, d, h, w = z.shape
    k_total = embedding.shape[0]
    flat_z = jnp.transpose(z, (0, 2, 3, 1)).reshape(-1, d)
    zsq = jnp.sum(flat_z ** 2, axis=1, keepdims=True)
    esq = jnp.sum(embedding ** 2, axis=1)

    bn = 2736
    k_pad = 3 * bn
    emb_pad = jnp.concatenate(
        [embedding, jnp.zeros((k_pad - k_total, d), embedding.dtype)])
    esq_pad = jnp.concatenate(
        [esq * 0.5, jnp.full((k_pad - k_total,), 1e30, esq.dtype)])
    encoding_indices = _fused_argmin(flat_z, emb_pad,
                                     (zsq * 0.5).T, esq_pad[:, None],
                                     bm=512, bn=bn)

    quantized_flat = jnp.take(embedding, encoding_indices, axis=0)
    counts = jnp.bincount(encoding_indices, length=k_total).astype(flat_z.dtype)

    z_q_flat = flat_z + (quantized_flat - flat_z)
    z_q = jnp.transpose(z_q_flat.reshape(b, h, w, d), (0, 3, 1, 2))
    loss = jnp.mean((z_q_flat - flat_z) ** 2)
    indices = encoding_indices.reshape(b, h, w)
    avg_probs = counts / (b * h * w)
    perplexity = jnp.exp(-jnp.sum(avg_probs * jnp.log(avg_probs + 1e-10)))
    used_codes = (counts > 0).astype(jnp.float32)
    return (z_q, loss, indices, perplexity, used_codes)
